# baseline (speedup 1.0000x reference)
import jax
import jax.numpy as jnp
from jax.experimental import pallas as pl

TB = 128


def _body(x_ref, srow_ref, rrow_ref, scol_ref, rcol_ref,
          w1_ref, b1_ref, w2_ref, b2_ref,
          cw1a_ref, cw1b_ref, cb1_ref, cw2_ref, cb2_ref,
          out_ref):
    f32 = jnp.float32
    x = x_ref[...]
    n = x.shape[1]
    p = srow_ref.shape[1]

    hi = jnp.tanh(x[:, :, None] * w1_ref[...][None, :, :]
                  + b1_ref[...][None, :, :])
    intr = jnp.sum(hi * w2_ref[...][None, :, :], axis=-1) + b2_ref[...]

    iota_np = jax.lax.broadcasted_iota(jnp.int32, (n, p), 0)
    gs = (iota_np == srow_ref[...]).astype(f32)
    gr = (iota_np == rrow_ref[...]).astype(f32)
    xs = jnp.dot(x, gs, preferred_element_type=f32)
    xr = jnp.dot(x, gr, preferred_element_type=f32)

    pre = (xs[:, :, None] * cw1a_ref[...][None, :, :]
           + xr[:, :, None] * cw1b_ref[...][None, :, :]
           + cb1_ref[...][None, :, :])
    hc = jnp.tanh(pre)
    contrib = jnp.sum(hc * cw2_ref[...][None, :, :], axis=-1) + cb2_ref[0, 0]

    iota_pn = jax.lax.broadcasted_iota(jnp.int32, (p, n), 1)
    s_mat = ((iota_pn == scol_ref[...]).astype(f32)
             - (iota_pn == rcol_ref[...]).astype(f32))
    coup = jnp.dot(contrib, s_mat, preferred_element_type=f32)

    out_ref[...] = intr + coup


def kernel(x, pins, W1, b1, W2, b2, cW1, cb1, cW2, cb2):
    B, N = x.shape
    P = pins.shape[0]
    H = cW1.shape[1]
    srow = pins[:, 0].reshape(1, P)
    rrow = pins[:, 1].reshape(1, P)
    scol = pins[:, 0:1]
    rcol = pins[:, 1:2]
    w1 = W1.reshape(N, H)
    w2 = W2.reshape(N, H)
    b2r = b2.reshape(1, N)
    cw1a = cW1[0:1, :]
    cw1b = cW1[1:2, :]
    cb1r = cb1.reshape(1, H)
    cw2r = cW2.reshape(1, H)
    cb2r = cb2.reshape(1, 1)

    full = lambda shape: pl.BlockSpec(shape, lambda i: (0,) * len(shape))
    return pl.pallas_call(
        _body,
        grid=(B // TB,),
        in_specs=[
            pl.BlockSpec((TB, N), lambda i: (i, 0)),
            full((1, P)), full((1, P)), full((P, 1)), full((P, 1)),
            full((N, H)), full((N, H)), full((N, H)), full((1, N)),
            full((1, H)), full((1, H)), full((1, H)), full((1, H)),
            full((1, 1)),
        ],
        out_specs=pl.BlockSpec((TB, N), lambda i: (i, 0)),
        out_shape=jax.ShapeDtypeStruct((B, N), x.dtype),
    )(x, srow, rrow, scol, rcol, w1, b1, w2, b2r, cw1a, cw1b, cb1r, cw2r,
      cb2r)


# speedup vs baseline: 1.7338x; 1.7338x over previous
import jax
import jax.numpy as jnp
from jax.experimental import pallas as pl
from jax.experimental.pallas import tpu as pltpu

TB = 256


def _body(x_ref, srow_ref, rrow_ref, scol_ref, rcol_ref,
          w1_ref, b1_ref, w2_ref, b2_ref,
          cw_ref, cb2_ref,
          out_ref):
    f32 = jnp.float32
    x = x_ref[...]
    n = x.shape[1]
    p = srow_ref.shape[1]
    hdim = w1_ref.shape[1]

    hi = jnp.tanh(x[:, :, None] * w1_ref[...][None, :, :]
                  + b1_ref[...][None, :, :])
    intr = jnp.sum(hi * w2_ref[...][None, :, :], axis=-1) + b2_ref[...]

    iota_np = jax.lax.broadcasted_iota(jnp.int32, (n, p), 0)
    gs = (iota_np == srow_ref[...]).astype(f32)
    gr = (iota_np == rrow_ref[...]).astype(f32)
    xs = jnp.dot(x, gs, preferred_element_type=f32)
    xr = jnp.dot(x, gr, preferred_element_type=f32)

    def h_step(h, acc):
        a = cw_ref[0, h]
        b = cw_ref[1, h]
        c = cw_ref[2, h]
        w = cw_ref[3, h]
        return acc + w * jnp.tanh(xs * a + xr * b + c)
    contrib = jax.lax.fori_loop(
        0, hdim, h_step, jnp.full((x.shape[0], p), cb2_ref[0, 0], dtype=f32),
        unroll=4)

    iota_pn = jax.lax.broadcasted_iota(jnp.int32, (p, n), 1)
    s_mat = ((iota_pn == scol_ref[...]).astype(f32)
             - (iota_pn == rcol_ref[...]).astype(f32))
    coup = jnp.dot(contrib, s_mat, preferred_element_type=f32)

    out_ref[...] = intr + coup


def kernel(x, pins, W1, b1, W2, b2, cW1, cb1, cW2, cb2):
    B, N = x.shape
    P = pins.shape[0]
    H = cW1.shape[1]
    srow = pins[:, 0].reshape(1, P)
    rrow = pins[:, 1].reshape(1, P)
    scol = pins[:, 0:1]
    rcol = pins[:, 1:2]
    w1 = W1.reshape(N, H)
    w2 = W2.reshape(N, H)
    b2r = b2.reshape(1, N)
    cw = jnp.concatenate([cW1[0:1, :], cW1[1:2, :],
                          cb1.reshape(1, H), cW2.reshape(1, H)], axis=0)
    cb2r = cb2.reshape(1, 1)

    full = lambda shape: pl.BlockSpec(shape, lambda i: (0,) * len(shape))
    return pl.pallas_call(
        _body,
        grid=(B // TB,),
        in_specs=[
            pl.BlockSpec((TB, N), lambda i: (i, 0)),
            full((1, P)), full((1, P)), full((P, 1)), full((P, 1)),
            full((N, H)), full((N, H)), full((N, H)), full((1, N)),
            pl.BlockSpec(memory_space=pltpu.SMEM),
            pl.BlockSpec(memory_space=pltpu.SMEM),
        ],
        out_specs=pl.BlockSpec((TB, N), lambda i: (i, 0)),
        out_shape=jax.ShapeDtypeStruct((B, N), x.dtype),
    )(x, srow, rrow, scol, rcol, w1, b1, w2, b2r, cw, cb2r)


# speedup vs baseline: 2.0850x; 1.2026x over previous
import jax
import jax.numpy as jnp
from jax.experimental import pallas as pl
from jax.experimental.pallas import tpu as pltpu

TB = 256
CH = 32


def _body(x_ref, srow_ref, rrow_ref, scol_ref, rcol_ref,
          w1t_ref, b1f_ref, w2f_ref, b2_ref, cw_ref, cb2_ref,
          out_ref, e_scr, r_scr, gs_scr, gr_scr, sm_scr):
    f32 = jnp.float32
    tb = x_ref.shape[0]
    n = x_ref.shape[1]
    p = srow_ref.shape[1]
    hdim = w1t_ref.shape[0]

    @pl.when(pl.program_id(0) == 0)
    def _init():
        iota_np = jax.lax.broadcasted_iota(jnp.int32, (n, p), 0)
        gs_scr[...] = (iota_np == srow_ref[...]).astype(f32)
        gr_scr[...] = (iota_np == rrow_ref[...]).astype(f32)
        iota_pn = jax.lax.broadcasted_iota(jnp.int32, (p, n), 1)
        sm_scr[...] = ((iota_pn == scol_ref[...]).astype(f32)
                       - (iota_pn == rcol_ref[...]).astype(f32))
        oh = (jax.lax.broadcasted_iota(jnp.int32, (n, n), 0)
              == jax.lax.broadcasted_iota(jnp.int32, (n, n), 1)).astype(f32)
        e3 = oh[:, None, :] * w1t_ref[...][None, :, :]
        e_scr[...] = e3.reshape(n, hdim * n)
        k_iota = jax.lax.broadcasted_iota(jnp.int32, (hdim * n, n), 0)
        i_iota = jax.lax.broadcasted_iota(jnp.int32, (hdim * n, n), 1)
        r_scr[...] = ((k_iota % n) == i_iota).astype(f32)

    x = x_ref[...]

    pre_i = (jnp.dot(x, e_scr[...], preferred_element_type=f32)
             + b1f_ref[...])
    ti = jnp.tanh(pre_i) * w2f_ref[...]
    intr = jnp.dot(ti, r_scr[...], preferred_element_type=f32) + b2_ref[...]

    xs = jnp.dot(x, gs_scr[...], preferred_element_type=f32)
    xr = jnp.dot(x, gr_scr[...], preferred_element_type=f32)

    cb2 = cb2_ref[0, 0]
    parts = []
    for c in range(tb // CH):
        xs_c = xs[c * CH:(c + 1) * CH, :]
        xr_c = xr[c * CH:(c + 1) * CH, :]

        def h_step(h, acc, xs_c=xs_c, xr_c=xr_c):
            a = cw_ref[0, h]
            b = cw_ref[1, h]
            cc = cw_ref[2, h]
            w = cw_ref[3, h]
            return acc + w * jnp.tanh(xs_c * a + xr_c * b + cc)

        parts.append(jax.lax.fori_loop(
            0, hdim, h_step, jnp.full((CH, p), cb2, dtype=f32), unroll=2))
    contrib = jnp.concatenate(parts, axis=0)

    coup = jnp.dot(contrib, sm_scr[...], preferred_element_type=f32)
    out_ref[...] = intr + coup


def kernel(x, pins, W1, b1, W2, b2, cW1, cb1, cW2, cb2):
    B, N = x.shape
    P = pins.shape[0]
    H = cW1.shape[1]
    f32 = jnp.float32
    srow = pins[:, 0].reshape(1, P)
    rrow = pins[:, 1].reshape(1, P)
    scol = pins[:, 0:1]
    rcol = pins[:, 1:2]
    w1t = W1.reshape(N, H).T
    b1f = b1.T.reshape(1, H * N)
    w2f = W2.reshape(N, H).T.reshape(1, H * N)
    b2r = b2.reshape(1, N)
    cw = jnp.concatenate([cW1[0:1, :], cW1[1:2, :],
                          cb1.reshape(1, H), cW2.reshape(1, H)], axis=0)
    cb2r = cb2.reshape(1, 1)

    full = lambda shape: pl.BlockSpec(shape, lambda i: (0,) * len(shape))
    return pl.pallas_call(
        _body,
        grid=(B // TB,),
        in_specs=[
            pl.BlockSpec((TB, N), lambda i: (i, 0)),
            full((1, P)), full((1, P)), full((P, 1)), full((P, 1)),
            full((H, N)), full((1, H * N)), full((1, H * N)), full((1, N)),
            pl.BlockSpec(memory_space=pltpu.SMEM),
            pl.BlockSpec(memory_space=pltpu.SMEM),
        ],
        out_specs=pl.BlockSpec((TB, N), lambda i: (i, 0)),
        out_shape=jax.ShapeDtypeStruct((B, N), x.dtype),
        scratch_shapes=[
            pltpu.VMEM((N, H * N), f32),
            pltpu.VMEM((H * N, N), f32),
            pltpu.VMEM((N, P), f32),
            pltpu.VMEM((N, P), f32),
            pltpu.VMEM((P, N), f32),
        ],
    )(x, srow, rrow, scol, rcol, w1t, b1f, w2f, b2r, cw, cb2r)
